# baseline (speedup 1.0000x reference)
import functools

import jax
import jax.numpy as jnp
from jax import lax
from jax.experimental import pallas as pl
from jax.experimental.pallas import tpu as pltpu
from jax.experimental.pallas import tpu_sc as plsc

NC = 2
NS = 16
CHUNK = 128


@functools.lru_cache(maxsize=None)
def _make_agg(n_nodes: int, dh: int, cpw: int, rpt: int):
    np_rows = NS * rpt
    mesh = plsc.VectorSubcoreMesh(core_axis_name="c", subcore_axis_name="s")

    @functools.partial(
        pl.kernel,
        out_type=jax.ShapeDtypeStruct((np_rows, NC * dh), jnp.float32),
        mesh=mesh,
        scratch_types=[
            pltpu.VMEM((cpw, CHUNK), jnp.int32),
            pltpu.VMEM((cpw, CHUNK), jnp.int32),
            pltpu.VMEM((2, CHUNK, dh), jnp.float32),
            pltpu.VMEM_SHARED((np_rows, dh), jnp.float32),
            pltpu.SemaphoreType.DMA,
            pltpu.SemaphoreType.DMA,
        ],
        compiler_params=pltpu.CompilerParams(use_tc_tiling_on_sc=False),
    )
    def agg(hs_hbm, srcw_hbm, dstw_hbm, zeros_hbm, out_hbm,
            src_v, dst_v, rows_v, acc_sh, sem0, sem1):
        c = lax.axis_index("c")
        s = lax.axis_index("s")
        col = pl.ds(c * dh, dh)
        h_my = hs_hbm.at[c]
        pltpu.sync_copy(zeros_hbm, acc_sh.at[pl.ds(s * rpt, rpt)])
        pltpu.sync_copy(srcw_hbm.at[s], src_v)
        pltpu.sync_copy(dstw_hbm.at[s], dst_v)
        plsc.subcore_barrier()

        sems = (sem0, sem1)
        for b in range(2):
            pltpu.async_copy(h_my.at[src_v.at[b]], rows_v.at[b], sems[b])

        def body(i, carry):
            j2 = i * 2
            for b in range(2):
                j = j2 + b
                pltpu.make_async_copy(
                    h_my.at[src_v.at[j]], rows_v.at[b], sems[b]).wait()
                pltpu.sync_copy(rows_v.at[b], acc_sh.at[dst_v.at[j]], add=True)

                @pl.when(j + 2 < cpw)
                def _():
                    pltpu.async_copy(
                        h_my.at[src_v.at[j + 2]], rows_v.at[b], sems[b])
            return carry

        lax.fori_loop(0, cpw // 2, body, 0)
        plsc.subcore_barrier()
        pltpu.sync_copy(acc_sh.at[pl.ds(s * rpt, rpt)],
                        out_hbm.at[pl.ds(s * rpt, rpt), col])

    return agg


def _stage1_body(a_ref, h_ref, w1_ref, b1_ref, sc_ref, z_ref, sums_ref):
    k = pl.program_id(0)
    a = a_ref[...]
    an = a / jnp.maximum(
        jnp.sqrt(jnp.sum(a * a, axis=1, keepdims=True)), 1e-12)
    h = h_ref[...]
    hn = h / jnp.maximum(
        jnp.sqrt(jnp.sum(h * h, axis=1, keepdims=True)), 1e-12)
    out = an + sc_ref[...] * hn
    z = jnp.dot(out, w1_ref[...], preferred_element_type=jnp.float32)
    z = z + b1_ref[...]
    z_ref[...] = z

    @pl.when(k == 0)
    def _():
        sums_ref[...] = jnp.zeros_like(sums_ref)

    sums_ref[0:1, :] += jnp.sum(z, axis=0, keepdims=True)
    sums_ref[1:2, :] += jnp.sum(z * z, axis=0, keepdims=True)


def _stage2_body(z_ref, sums_ref, g_ref, be_ref, w2_ref, b2_ref, out_ref,
                 *, n_rows, relu_out):
    mu = sums_ref[0:1, :] / n_rows
    var = sums_ref[1:2, :] / n_rows - mu * mu
    inv = g_ref[...] / jnp.sqrt(var + 1e-5)
    z = (z_ref[...] - mu) * inv + be_ref[...]
    z = jnp.maximum(z, 0.0)
    hnew = jnp.dot(z, w2_ref[...], preferred_element_type=jnp.float32)
    hnew = hnew + b2_ref[...]
    if relu_out:
        hnew = jnp.maximum(hnew, 0.0)
    out_ref[...] = hnew


def _row_block(n):
    for b in (10000, 5000, 2000, 1000, 500, 250, 200, 100, 40, 8):
        if n % b == 0 and b % 8 == 0:
            return b
    return 8


def kernel(x, edge_index, W1, b1, gamma, beta, W2, b2, eps):
    n, d = x.shape
    dh = d // 2
    n_layers = W1.shape[0]
    e = edge_index.shape[1]

    cpw = -(-e // (NS * CHUNK))
    cpw += cpw % 2
    ep = cpw * NS * CHUNK
    pad = ep - e
    rpt = -(-(n + 1) // NS)
    rpt = -(-rpt // 8) * 8

    src = jnp.concatenate([edge_index[0], jnp.zeros((pad,), jnp.int32)])
    dst = jnp.concatenate([edge_index[1], jnp.full((pad,), n, jnp.int32)])
    srcw = src.reshape(NS, cpw, CHUNK)
    dstw = dst.reshape(NS, cpw, CHUNK)
    zeros = jnp.zeros((rpt, dh), jnp.float32)

    agg_fn = _make_agg(n, dh, cpw, rpt)

    b_rows = _row_block(n)
    grid = n // b_rows
    row_spec = pl.BlockSpec((b_rows, d), lambda k: (k, 0))
    mat_spec = pl.BlockSpec((d, d), lambda k: (0, 0))
    vec_spec = pl.BlockSpec((1, d), lambda k: (0, 0))
    sums_spec = pl.BlockSpec((8, d), lambda k: (0, 0))

    stage1 = pl.pallas_call(
        _stage1_body,
        grid=(grid,),
        in_specs=[row_spec, row_spec, mat_spec, vec_spec, vec_spec],
        out_specs=[row_spec, sums_spec],
        out_shape=[jax.ShapeDtypeStruct((n, d), jnp.float32),
                   jax.ShapeDtypeStruct((8, d), jnp.float32)],
    )

    h = x
    for i in range(n_layers):
        hs = jnp.stack([h[:, :dh], h[:, dh:]])
        agg = agg_fn(hs, srcw, dstw, zeros)
        scale_row = jnp.full((1, d), 1.0 + eps[i], jnp.float32)
        z, sums = stage1(agg, h, W1[i], b1[i].reshape(1, d), scale_row)
        stage2 = pl.pallas_call(
            functools.partial(_stage2_body, n_rows=float(n),
                              relu_out=(i != n_layers - 1)),
            grid=(grid,),
            in_specs=[row_spec, sums_spec, vec_spec, vec_spec, mat_spec,
                      vec_spec],
            out_specs=row_spec,
            out_shape=jax.ShapeDtypeStruct((n, d), jnp.float32),
        )
        h = stage2(z, sums, gamma[i].reshape(1, d), beta[i].reshape(1, d),
                   W2[i], b2[i].reshape(1, d))
    return h


# speedup vs baseline: 1.0394x; 1.0394x over previous
import functools

import jax
import jax.numpy as jnp
from jax import lax
from jax.experimental import pallas as pl
from jax.experimental.pallas import tpu as pltpu
from jax.experimental.pallas import tpu_sc as plsc

NC = 2
NS = 16
CHUNK = 128


@functools.lru_cache(maxsize=None)
def _make_agg(n_nodes: int, dh: int, cpw: int, rpt: int):
    np_rows = NS * rpt
    mesh = plsc.VectorSubcoreMesh(core_axis_name="c", subcore_axis_name="s")

    @functools.partial(
        pl.kernel,
        out_type=jax.ShapeDtypeStruct((np_rows, NC * dh), jnp.float32),
        mesh=mesh,
        scratch_types=[
            pltpu.VMEM((cpw, CHUNK), jnp.int32),
            pltpu.VMEM((cpw, CHUNK), jnp.int32),
            pltpu.VMEM((2, CHUNK, dh), jnp.float32),
            pltpu.VMEM_SHARED((np_rows, dh), jnp.float32),
            pltpu.SemaphoreType.DMA,
            pltpu.SemaphoreType.DMA,
        ],
        compiler_params=pltpu.CompilerParams(use_tc_tiling_on_sc=False),
    )
    def agg(hs_hbm, srcw_hbm, dstw_hbm, zeros_hbm, out_hbm,
            src_v, dst_v, rows_v, acc_sh, sem0, sem1):
        c = lax.axis_index("c")
        s = lax.axis_index("s")
        col = pl.ds(c * dh, dh)
        h_my = hs_hbm.at[c]
        pltpu.sync_copy(zeros_hbm, acc_sh.at[pl.ds(s * rpt, rpt)])
        pltpu.sync_copy(srcw_hbm.at[s], src_v)
        pltpu.sync_copy(dstw_hbm.at[s], dst_v)
        plsc.subcore_barrier()

        sems = (sem0, sem1)
        for b in range(2):
            pltpu.async_copy(h_my.at[src_v.at[b]], rows_v.at[b], sems[b])

        def body(i, carry):
            j2 = i * 2
            for b in range(2):
                j = j2 + b
                pltpu.make_async_copy(
                    h_my.at[src_v.at[j]], rows_v.at[b], sems[b]).wait()
                pltpu.sync_copy(rows_v.at[b], acc_sh.at[dst_v.at[j]], add=True)

                @pl.when(j + 2 < cpw)
                def _():
                    pltpu.async_copy(
                        h_my.at[src_v.at[j + 2]], rows_v.at[b], sems[b])
            return carry

        lax.fori_loop(0, cpw // 2, body, 0)
        plsc.subcore_barrier()
        pltpu.sync_copy(acc_sh.at[pl.ds(s * rpt, rpt)],
                        out_hbm.at[pl.ds(s * rpt, rpt), col])

    return agg


def _stage1_body(a_ref, h_ref, w1_ref, b1_ref, sc_ref, z_ref, sums_ref):
    k = pl.program_id(0)
    a = a_ref[...]
    an = a / jnp.maximum(
        jnp.sqrt(jnp.sum(a * a, axis=1, keepdims=True)), 1e-12)
    h = h_ref[...]
    hn = h / jnp.maximum(
        jnp.sqrt(jnp.sum(h * h, axis=1, keepdims=True)), 1e-12)
    out = an + sc_ref[...] * hn
    z = jnp.dot(out, w1_ref[...], preferred_element_type=jnp.float32)
    z = z + b1_ref[...]
    z_ref[...] = z

    @pl.when(k == 0)
    def _():
        sums_ref[...] = jnp.zeros_like(sums_ref)

    sums_ref[0:1, :] += jnp.sum(z, axis=0, keepdims=True)
    sums_ref[1:2, :] += jnp.sum(z * z, axis=0, keepdims=True)


def _stage2_body(z_ref, sums_ref, g_ref, be_ref, w2_ref, b2_ref, out_ref,
                 *, n_rows, relu_out):
    mu = sums_ref[0:1, :] / n_rows
    var = sums_ref[1:2, :] / n_rows - mu * mu
    inv = g_ref[...] / jnp.sqrt(var + 1e-5)
    z = (z_ref[...] - mu) * inv + be_ref[...]
    z = jnp.maximum(z, 0.0)
    hnew = jnp.dot(z, w2_ref[...], preferred_element_type=jnp.float32)
    hnew = hnew + b2_ref[...]
    if relu_out:
        hnew = jnp.maximum(hnew, 0.0)
    out_ref[...] = hnew


def _row_block(n):
    for b in (5000, 2000, 1000, 500, 250, 200, 100, 40, 8):
        if n % b == 0 and b % 8 == 0:
            return b
    return 8


def kernel(x, edge_index, W1, b1, gamma, beta, W2, b2, eps):
    n, d = x.shape
    dh = d // 2
    n_layers = W1.shape[0]
    e = edge_index.shape[1]

    cpw = -(-e // (NS * CHUNK))
    cpw += cpw % 2
    ep = cpw * NS * CHUNK
    pad = ep - e
    rpt = -(-(n + 1) // NS)
    rpt = -(-rpt // 8) * 8

    src = jnp.concatenate([edge_index[0], jnp.zeros((pad,), jnp.int32)])
    dst = jnp.concatenate([edge_index[1], jnp.full((pad,), n, jnp.int32)])
    srcw = src.reshape(NS, cpw, CHUNK)
    dstw = dst.reshape(NS, cpw, CHUNK)
    zeros = jnp.zeros((rpt, dh), jnp.float32)

    agg_fn = _make_agg(n, dh, cpw, rpt)

    b_rows = _row_block(n)
    grid = n // b_rows
    row_spec = pl.BlockSpec((b_rows, d), lambda k: (k, 0))
    mat_spec = pl.BlockSpec((d, d), lambda k: (0, 0))
    vec_spec = pl.BlockSpec((1, d), lambda k: (0, 0))
    sums_spec = pl.BlockSpec((8, d), lambda k: (0, 0))

    stage1 = pl.pallas_call(
        _stage1_body,
        grid=(grid,),
        in_specs=[row_spec, row_spec, mat_spec, vec_spec, vec_spec],
        out_specs=[row_spec, sums_spec],
        out_shape=[jax.ShapeDtypeStruct((n, d), jnp.float32),
                   jax.ShapeDtypeStruct((8, d), jnp.float32)],
    )

    h = x
    for i in range(n_layers):
        hs = jnp.stack([h[:, :dh], h[:, dh:]])
        agg = agg_fn(hs, srcw, dstw, zeros)
        scale_row = jnp.full((1, d), 1.0 + eps[i], jnp.float32)
        z, sums = stage1(agg, h, W1[i], b1[i].reshape(1, d), scale_row)
        stage2 = pl.pallas_call(
            functools.partial(_stage2_body, n_rows=float(n),
                              relu_out=(i != n_layers - 1)),
            grid=(grid,),
            in_specs=[row_spec, sums_spec, vec_spec, vec_spec, mat_spec,
                      vec_spec],
            out_specs=row_spec,
            out_shape=jax.ShapeDtypeStruct((n, d), jnp.float32),
        )
        h = stage2(z, sums, gamma[i].reshape(1, d), beta[i].reshape(1, d),
                   W2[i], b2[i].reshape(1, d))
    return h
